# Initial kernel scaffold; baseline (speedup 1.0000x reference)
import jax
import jax.numpy as jnp
from jax.experimental import pallas as pl


def kernel(x, W_enc, b_enc, W_dec, b_dec):
    raise NotImplementedError("write your pallas kernel here")


# speedup vs baseline: 17.7535x; 17.7535x over previous
import functools

import jax
import jax.numpy as jnp
from jax.experimental import pallas as pl
from jax.experimental.pallas import tpu as pltpu

_K = 64
_ROWS = 256

def _body(x_ref, we_ref, be_ref, wd_ref, bd_ref, pre_ref, sp_ref, rec_ref,
          *, k):
    xb = x_ref[...]
    pre = jax.lax.dot_general(
        xb, we_ref[...], (((1,), (1,)), ((), ())),
        preferred_element_type=jnp.float32) + be_ref[...]
    pre_ref[...] = pre

    s = jax.lax.bitcast_convert_type(pre, jnp.int32)
    ks = jnp.where(s >= 0, s, s ^ jnp.int32(0x7FFFFFFF))

    cand = jnp.full((pre.shape[0], 1), -(2**31), dtype=jnp.int32)
    for bit in range(31, -1, -1):
        inc = jnp.int32(-(2**31) if bit == 31 else 1 << bit)
        t = cand + inc
        cnt = jnp.sum((ks >= t).astype(jnp.int32), axis=1, keepdims=True)
        cand = jnp.where(cnt >= k, t, cand)

    sp = jnp.where(ks >= cand, pre, 0.0)
    sp_ref[...] = sp
    rec_ref[...] = jax.lax.dot_general(
        sp, wd_ref[...], (((1,), (1,)), ((), ())),
        preferred_element_type=jnp.float32) + bd_ref[...]


def kernel(x, W_enc, b_enc, W_dec, b_dec):
    n, d = x.shape
    f = W_enc.shape[0]
    r = _ROWS if n % _ROWS == 0 else n
    grid = (n // r,)

    out = pl.pallas_call(
        functools.partial(_body, k=_K),
        grid=grid,
        in_specs=[
            pl.BlockSpec((r, d), lambda i: (i, 0)),
            pl.BlockSpec((f, d), lambda i: (0, 0)),
            pl.BlockSpec((1, f), lambda i: (0, 0)),
            pl.BlockSpec((d, f), lambda i: (0, 0)),
            pl.BlockSpec((1, d), lambda i: (0, 0)),
        ],
        out_specs=[
            pl.BlockSpec((r, f), lambda i: (i, 0)),
            pl.BlockSpec((r, f), lambda i: (i, 0)),
            pl.BlockSpec((r, d), lambda i: (i, 0)),
        ],
        out_shape=[
            jax.ShapeDtypeStruct((n, f), jnp.float32),
            jax.ShapeDtypeStruct((n, f), jnp.float32),
            jax.ShapeDtypeStruct((n, d), jnp.float32),
        ],
        compiler_params=pltpu.CompilerParams(
            dimension_semantics=("arbitrary",),
        ),
    )(x, W_enc, b_enc.reshape(1, f), W_dec, b_dec.reshape(1, d))
    pre_acts, sparse_acts, reconstruction = out
    return (reconstruction, sparse_acts, pre_acts)
